# baseline (speedup 1.0000x reference)
import functools

import jax
import jax.numpy as jnp
from jax.experimental import pallas as pl
from jax.experimental.pallas import tpu as pltpu

N_ATOMS = 320000
ATOM_FDIM = 128
HIDDEN = 128
NUM_MOLS = 10000

BLK = 12800
RSEG = 512
ACC_ROWS = NUM_MOLS + RSEG


def _fused_kernel(ids_ref, f_ref, wt_ref, b_ref, out_ref, acc_ref, cnt_ref):
    k = pl.program_id(0)
    nsteps = pl.num_programs(0)

    @pl.when(k == 0)
    def _init():
        acc_ref[...] = jnp.zeros_like(acc_ref)
        cnt_ref[...] = jnp.zeros_like(cnt_ref)

    h = jnp.dot(f_ref[...], wt_ref[...], preferred_element_type=jnp.float32)
    h = jnp.maximum(h + b_ref[...], 0.0)

    ids = ids_ref[0, 0, :]
    base0 = ids[0] & ~7
    span = ids[BLK - 1] - base0 + 1
    nchunks = (span + RSEG - 1) // RSEG
    ids_row = ids.reshape(1, BLK)
    riota = jax.lax.broadcasted_iota(jnp.int32, (RSEG, BLK), 0)

    def chunk(j, _):
        base = base0 + j * RSEG
        p = (ids_row - base == riota).astype(jnp.float32)
        psum = jnp.dot(p, h, preferred_element_type=jnp.float32)
        pcnt = jnp.sum(p, axis=1)
        acc_ref[pl.ds(base, RSEG), :] += psum
        cnt_ref[pl.ds(base, RSEG), :] += pcnt[:, None]
        return _

    chunk(0, None)

    @pl.when(nchunks > 1)
    def _rest():
        jax.lax.fori_loop(1, nchunks, chunk, None)

    @pl.when(k == nsteps - 1)
    def _finish():
        acc = acc_ref[: NUM_MOLS, :]
        cnt = cnt_ref[: NUM_MOLS, :]
        out_ref[...] = jnp.where(cnt > 0.0, acc / jnp.maximum(cnt, 1.0), 0.0)


@jax.jit
def kernel(f_atoms, segment_ids, W, b):
    nblk = N_ATOMS // BLK
    ids3 = segment_ids.reshape(nblk, 1, BLK)
    wt = W.T
    b_row = b.reshape(1, HIDDEN)
    return pl.pallas_call(
        _fused_kernel,
        grid=(nblk,),
        in_specs=[
            pl.BlockSpec((1, 1, BLK), lambda k: (k, 0, 0)),
            pl.BlockSpec((BLK, ATOM_FDIM), lambda k: (k, 0)),
            pl.BlockSpec((ATOM_FDIM, HIDDEN), lambda k: (0, 0)),
            pl.BlockSpec((1, HIDDEN), lambda k: (0, 0)),
        ],
        out_specs=pl.BlockSpec((NUM_MOLS, HIDDEN), lambda k: (0, 0)),
        out_shape=jax.ShapeDtypeStruct((NUM_MOLS, HIDDEN), jnp.float32),
        scratch_shapes=[
            pltpu.VMEM((ACC_ROWS, HIDDEN), jnp.float32),
            pltpu.VMEM((ACC_ROWS, HIDDEN), jnp.float32),
        ],
    )(ids3, f_atoms, wt, b_row)


# speedup vs baseline: 1.5378x; 1.5378x over previous
import functools

import jax
import jax.numpy as jnp
from jax.experimental import pallas as pl
from jax.experimental.pallas import tpu as pltpu

N_ATOMS = 320000
ATOM_FDIM = 128
HIDDEN = 128
NUM_MOLS = 10000

BLK = 6400
RSEG = 256
ACC_ROWS = NUM_MOLS + RSEG


def _fused_kernel(ids_ref, f_ref, wt_ref, b_ref, out_ref, acc_ref, cnt_ref):
    k = pl.program_id(0)
    nsteps = pl.num_programs(0)

    @pl.when(k == 0)
    def _init():
        acc_ref[...] = jnp.zeros_like(acc_ref)
        cnt_ref[...] = jnp.zeros_like(cnt_ref)

    h = jnp.dot(f_ref[...], wt_ref[...], preferred_element_type=jnp.float32)
    h = jnp.maximum(h + b_ref[...], 0.0)

    ids = ids_ref[0, 0, :]
    base0 = ids[0] & ~7
    span = ids[BLK - 1] - base0 + 1
    nchunks = (span + RSEG - 1) // RSEG
    ids_row = ids.reshape(1, BLK)
    riota = jax.lax.broadcasted_iota(jnp.int32, (RSEG, BLK), 0)

    hb = h.astype(jnp.bfloat16)

    def chunk(j, _):
        base = base0 + j * RSEG
        p = (ids_row - base == riota).astype(jnp.bfloat16)
        psum = jnp.dot(p, hb, preferred_element_type=jnp.float32)
        pcnt = jnp.sum(p.astype(jnp.float32), axis=1)
        acc_ref[pl.ds(base, RSEG), :] += psum
        cnt_ref[pl.ds(base, RSEG), :] += pcnt[:, None]
        return _

    chunk(0, None)

    @pl.when(nchunks > 1)
    def _rest():
        jax.lax.fori_loop(1, nchunks, chunk, None)

    @pl.when(k == nsteps - 1)
    def _finish():
        acc = acc_ref[: NUM_MOLS, :]
        cnt = cnt_ref[: NUM_MOLS, :]
        out_ref[...] = jnp.where(cnt > 0.0, acc / jnp.maximum(cnt, 1.0), 0.0)


@jax.jit
def kernel(f_atoms, segment_ids, W, b):
    nblk = N_ATOMS // BLK
    ids3 = segment_ids.reshape(nblk, 1, BLK)
    wt = W.T
    b_row = b.reshape(1, HIDDEN)
    return pl.pallas_call(
        _fused_kernel,
        grid=(nblk,),
        in_specs=[
            pl.BlockSpec((1, 1, BLK), lambda k: (k, 0, 0)),
            pl.BlockSpec((BLK, ATOM_FDIM), lambda k: (k, 0)),
            pl.BlockSpec((ATOM_FDIM, HIDDEN), lambda k: (0, 0)),
            pl.BlockSpec((1, HIDDEN), lambda k: (0, 0)),
        ],
        out_specs=pl.BlockSpec((NUM_MOLS, HIDDEN), lambda k: (0, 0)),
        out_shape=jax.ShapeDtypeStruct((NUM_MOLS, HIDDEN), jnp.float32),
        scratch_shapes=[
            pltpu.VMEM((ACC_ROWS, HIDDEN), jnp.float32),
            pltpu.VMEM((ACC_ROWS, HIDDEN), jnp.float32),
        ],
    )(ids3, f_atoms, wt, b_row)


# speedup vs baseline: 2.1418x; 1.3927x over previous
import jax
import jax.numpy as jnp
from jax.experimental import pallas as pl
from jax.experimental.pallas import tpu as pltpu

N_ATOMS = 320000
ATOM_FDIM = 128
HIDDEN = 128
NUM_MOLS = 10000

BLK = 16000
RSEG = 128
SUB = 3200
ACC_ROWS = NUM_MOLS + RSEG


def _fused_kernel(ids_ref, f_ref, wt_ref, b_ref, out_ref, acc_ref, cnt_ref):
    k = pl.program_id(0)
    nsteps = pl.num_programs(0)

    @pl.when(k == 0)
    def _init():
        acc_ref[...] = jnp.zeros_like(acc_ref)
        cnt_ref[...] = jnp.zeros_like(cnt_ref)

    h = jnp.dot(f_ref[...], wt_ref[...], preferred_element_type=jnp.float32)
    h = jnp.maximum(h + b_ref[...], 0.0)

    ids = ids_ref[0, 0, :]
    riota = jax.lax.broadcasted_iota(jnp.int32, (RSEG, SUB), 0)

    for s in range(BLK // SUB):
        ids_s = ids[s * SUB:(s + 1) * SUB]
        h_s = h[s * SUB:(s + 1) * SUB, :]
        base0 = ids_s[0] & ~7
        span = ids_s[SUB - 1] - base0 + 1
        nchunks = (span + RSEG - 1) // RSEG
        ids_row = ids_s.reshape(1, SUB)

        def chunk(j, _, ids_row=ids_row, h_s=h_s, base0=base0):
            base = base0 + j * RSEG
            p = (ids_row - base == riota).astype(jnp.float32)
            psum = jnp.dot(p, h_s, preferred_element_type=jnp.float32)
            pcnt = jnp.sum(p, axis=1)
            acc_ref[pl.ds(base, RSEG), :] += psum
            cnt_ref[pl.ds(base, RSEG), :] += pcnt[:, None]
            return _

        chunk(0, None)

        @pl.when(nchunks > 1)
        def _rest(chunk=chunk, nchunks=nchunks):
            jax.lax.fori_loop(1, nchunks, chunk, None)

    @pl.when(k == nsteps - 1)
    def _finish():
        acc = acc_ref[: NUM_MOLS, :]
        cnt = cnt_ref[: NUM_MOLS, :]
        out_ref[...] = jnp.where(cnt > 0.0, acc / jnp.maximum(cnt, 1.0), 0.0)


@jax.jit
def kernel(f_atoms, segment_ids, W, b):
    nblk = N_ATOMS // BLK
    ids3 = segment_ids.reshape(nblk, 1, BLK)
    wt = W.T
    b_row = b.reshape(1, HIDDEN)
    return pl.pallas_call(
        _fused_kernel,
        grid=(nblk,),
        in_specs=[
            pl.BlockSpec((1, 1, BLK), lambda k: (k, 0, 0)),
            pl.BlockSpec((BLK, ATOM_FDIM), lambda k: (k, 0)),
            pl.BlockSpec((ATOM_FDIM, HIDDEN), lambda k: (0, 0)),
            pl.BlockSpec((1, HIDDEN), lambda k: (0, 0)),
        ],
        out_specs=pl.BlockSpec((NUM_MOLS, HIDDEN), lambda k: (0, 0)),
        out_shape=jax.ShapeDtypeStruct((NUM_MOLS, HIDDEN), jnp.float32),
        scratch_shapes=[
            pltpu.VMEM((ACC_ROWS, HIDDEN), jnp.float32),
            pltpu.VMEM((ACC_ROWS, HIDDEN), jnp.float32),
        ],
    )(ids3, f_atoms, wt, b_row)
